# Initial kernel scaffold; baseline (speedup 1.0000x reference)
import jax
import jax.numpy as jnp
from jax.experimental import pallas as pl


def kernel(x, edge_index, edge_attr, conv_params, lin_params):
    raise NotImplementedError("write your pallas kernel here")


# speedup vs baseline: 3.2164x; 3.2164x over previous
import functools

import jax
import jax.numpy as jnp
from jax import lax
from jax.experimental import pallas as pl
from jax.experimental.pallas import tpu as pltpu
from jax.experimental.pallas import tpu_sc as plsc

N = 4096
E = 65536
NF = N * N


_CHUNK = 256
_NCHUNK = N // _CHUNK
_SH_WORDS = _CHUNK * N
_TPS = 16
_EPT = E // _TPS
_SLICE = _SH_WORDS // _TPS
_ZW = 16384
_SROW = 128
_NSROW = _EPT // _SROW + 1


def _sc_body(src_hbm, dst_hbm, w_hbm, out_hbm, s_buf, d_buf, w_buf,
             idx_buf, val_buf, zero_buf, sh, sem):
    c = lax.axis_index("c")
    t = lax.axis_index("s")
    base = t * _EPT
    pltpu.sync_copy(src_hbm.at[pl.ds(base, _EPT)], s_buf)
    pltpu.sync_copy(dst_hbm.at[pl.ds(base, _EPT)], d_buf)
    pltpu.sync_copy(w_hbm.at[pl.ds(base, _EPT)], w_buf)

    def zfill(i, carry):
        zero_buf[pl.ds(i * 16, 16)] = jnp.zeros((16,), jnp.float32)
        return carry
    lax.fori_loop(0, _ZW // 16, zfill, 0)

    lane = lax.iota(jnp.int32, 16)
    ones = jnp.ones((16,), jnp.float32)

    if True:
        for k in range(_NCHUNK // 2):
            chunk = 2 * k + c
            r0 = chunk * _CHUNK
            for j in range(_SLICE // _ZW):
                pltpu.sync_copy(
                    zero_buf, sh.at[pl.ds(t * _SLICE + j * _ZW, _ZW)])
            plsc.subcore_barrier()

            def emit(row, carry):
                for j in range(_SROW // 16):
                    sl = pl.ds(row * _SROW + j * 16, 16)
                    sv = s_buf[sl]
                    dv = d_buf[sl]
                    wv = w_buf[sl]
                    rel = dv - r0
                    ok = (rel >= 0) & (rel < _CHUNK)
                    dsl = pl.ds(j * 16, 16)
                    idx_buf[row, dsl] = jnp.where(ok, rel * N + sv, 0)
                    val_buf[row, dsl] = jnp.where(
                        ok, wv, jnp.zeros((16,), jnp.float32))
                return carry
            lax.fori_loop(0, _EPT // _SROW, emit, 0)
            drow = t * 16 + lane
            idx_buf[_NSROW - 1, pl.ds(0, 16)] = drow * N + r0 + drow
            val_buf[_NSROW - 1, pl.ds(0, 16)] = ones
            for j in range(1, _SROW // 16):
                idx_buf[_NSROW - 1, pl.ds(j * 16, 16)] = lane * 0
                val_buf[_NSROW - 1, pl.ds(j * 16, 16)] = ones * 0.0

            copies = [
                pltpu.async_copy(val_buf.at[j], sh.at[idx_buf.at[j]],
                                 sem, add=True)
                for j in range(_NSROW)
            ]
            for cp in copies:
                cp.wait()
            plsc.subcore_barrier()

            pltpu.sync_copy(
                sh.at[pl.ds(t * _SLICE, _SLICE)],
                out_hbm.at[pl.ds(chunk * _SH_WORDS + t * _SLICE, _SLICE)])


def _build_b(src, dst, w):
    fn = pl.kernel(
        _sc_body,
        out_type=jax.ShapeDtypeStruct((NF,), jnp.float32),
        mesh=plsc.VectorSubcoreMesh(core_axis_name="c", subcore_axis_name="s"),
        scratch_types=[
            pltpu.VMEM((_EPT,), jnp.int32),
            pltpu.VMEM((_EPT,), jnp.int32),
            pltpu.VMEM((_EPT,), jnp.float32),
            pltpu.VMEM((_NSROW, _SROW), jnp.int32),
            pltpu.VMEM((_NSROW, _SROW), jnp.float32),
            pltpu.VMEM((_ZW,), jnp.float32),
            pltpu.VMEM_SHARED((_SH_WORDS,), jnp.float32),
            pltpu.SemaphoreType.DMA,
        ],
    )
    return fn(src, dst, w)


def _dinv_kernel(b_ref, o_ref):
    deg = jnp.sum(b_ref[...], axis=1, keepdims=True)
    o_ref[...] = jnp.where(deg > 0, lax.rsqrt(deg), 0.0)


def _dinv(bmat):
    bm = 512
    return pl.pallas_call(
        _dinv_kernel,
        grid=(N // bm,),
        in_specs=[pl.BlockSpec((bm, N), lambda i: (i, 0))],
        out_specs=pl.BlockSpec((bm, 1), lambda i: (i, 0)),
        out_shape=jax.ShapeDtypeStruct((N, 1), jnp.float32),
    )(bmat)


def _mm_kernel(*refs, nk, kscale, mscale, mbias, nbias, lrelu, x3):
    it = iter(refs)
    x_ref = next(it)
    y_ref = next(it)
    ks_ref = next(it) if kscale else None
    ms_ref = next(it) if mscale else None
    mb_ref = next(it) if mbias else None
    nb_ref = next(it) if nbias else None
    o_ref = next(it)
    acc_ref = next(it)

    @pl.when(pl.program_id(2) == 0)
    def _():
        acc_ref[...] = jnp.zeros_like(acc_ref)

    yv = y_ref[...]
    if kscale:
        yv = yv * ks_ref[...]
    xv = x_ref[...]
    d = functools.partial(jnp.dot, preferred_element_type=jnp.float32)
    if x3 == 6:
        x1 = xv.astype(jnp.bfloat16)
        r = xv - x1.astype(jnp.float32)
        x2 = r.astype(jnp.bfloat16)
        x3_ = (r - x2.astype(jnp.float32)).astype(jnp.bfloat16)
        y1 = yv.astype(jnp.bfloat16)
        r = yv - y1.astype(jnp.float32)
        y2 = r.astype(jnp.bfloat16)
        y3_ = (r - y2.astype(jnp.float32)).astype(jnp.bfloat16)
        acc_ref[...] += (
            (d(x2, y2) + d(x1, y3_) + d(x3_, y1))
            + (d(x1, y2) + d(x2, y1)) + d(x1, y1))
    elif x3:
        xh = xv.astype(jnp.bfloat16)
        xl = (xv - xh.astype(jnp.float32)).astype(jnp.bfloat16)
        yh = yv.astype(jnp.bfloat16)
        yl = (yv - yh.astype(jnp.float32)).astype(jnp.bfloat16)
        acc_ref[...] += d(xh, yh) + (d(xh, yl) + d(xl, yh))
    else:
        acc_ref[...] += d(xv.astype(jnp.bfloat16), yv.astype(jnp.bfloat16))

    @pl.when(pl.program_id(2) == nk - 1)
    def _():
        r = acc_ref[...]
        if mscale:
            r = r * ms_ref[...]
        if nbias:
            r = r + nb_ref[...]
        if mbias:
            r = r + mb_ref[...]
        if lrelu:
            r = jnp.where(r > 0, r, 0.1 * r)
        o_ref[...] = r


def _mm(x, y, kscale=None, mscale=None, mbias=None, nbias=None,
        lrelu=False, x3=False, bm=512, bn=1024, bk=512):
    m, k = x.shape
    _, n = y.shape
    bm, bn, bk = min(bm, m), min(bn, n), min(bk, k)
    grid = (m // bm, n // bn, k // bk)
    in_specs = [
        pl.BlockSpec((bm, bk), lambda i, j, kk: (i, kk)),
        pl.BlockSpec((bk, bn), lambda i, j, kk: (kk, j)),
    ]
    args = [x, y]
    if kscale is not None:
        in_specs.append(pl.BlockSpec((bk, 1), lambda i, j, kk: (kk, 0)))
        args.append(kscale)
    if mscale is not None:
        in_specs.append(pl.BlockSpec((bm, 1), lambda i, j, kk: (i, 0)))
        args.append(mscale)
    if mbias is not None:
        in_specs.append(pl.BlockSpec((bm, 1), lambda i, j, kk: (i, 0)))
        args.append(mbias)
    if nbias is not None:
        in_specs.append(pl.BlockSpec((1, bn), lambda i, j, kk: (0, j)))
        args.append(nbias)
    body = functools.partial(
        _mm_kernel, nk=grid[2], kscale=kscale is not None,
        mscale=mscale is not None, mbias=mbias is not None,
        nbias=nbias is not None, lrelu=lrelu, x3=x3)
    return pl.pallas_call(
        body,
        grid=grid,
        in_specs=in_specs,
        out_specs=pl.BlockSpec((bm, bn), lambda i, j, kk: (i, j)),
        out_shape=jax.ShapeDtypeStruct((m, n), jnp.float32),
        scratch_shapes=[pltpu.VMEM((bm, bn), jnp.float32)],
        compiler_params=pltpu.CompilerParams(
            dimension_semantics=("parallel", "parallel", "arbitrary")),
    )(*args)


def _bn_kernel(y_ref, g_ref, b_ref, o_ref):
    yv = y_ref[...]
    m = jnp.mean(yv, axis=0, keepdims=True)
    v = jnp.mean((yv - m) ** 2, axis=0, keepdims=True)
    o_ref[...] = (yv - m) * lax.rsqrt(v + 1e-5) * g_ref[...] + b_ref[...]


def _bn(y, g, be):
    c = y.shape[1]
    bc = min(c, 256)
    return pl.pallas_call(
        _bn_kernel,
        grid=(c // bc,),
        in_specs=[
            pl.BlockSpec((N, bc), lambda j: (0, j)),
            pl.BlockSpec((1, bc), lambda j: (0, j)),
            pl.BlockSpec((1, bc), lambda j: (0, j)),
        ],
        out_specs=pl.BlockSpec((N, bc), lambda j: (0, j)),
        out_shape=jax.ShapeDtypeStruct((N, c), jnp.float32),
    )(y, g.reshape(1, c), be.reshape(1, c))


def kernel(x, edge_index, edge_attr, conv_params, lin_params):
    src = edge_index[0]
    dst = edge_index[1]
    bmat = _build_b(src, dst, edge_attr).reshape(N, N)
    dinv = _dinv(bmat)

    h = x
    for (w, b, g, be) in conv_params:
        ci, co = w.shape
        u = _mm(h, w)
        y = _mm(bmat, u, kscale=dinv, mscale=dinv,
                nbias=b.reshape(1, co), lrelu=True, x3=6)
        h = _bn(y, g, be)

    ht = h.reshape(32, -1).T
    n_lin = len(lin_params)
    for i, (w, b) in enumerate(lin_params):
        co = w.shape[0]
        ht = _mm(w, ht, mbias=b.reshape(co, 1), lrelu=(i < n_lin - 1),
                 bn=32)
    return ht.T
